# baseline (speedup 1.0000x reference)
import jax
import jax.numpy as jnp
from jax.experimental import pallas as pl
from jax.experimental.pallas import tpu as pltpu

B, N, L, D = 8, 9, 512, 768
S = N - 1


def _select_kernel(valid_ref, reps_in, mask_in, reps_out, mask_out):
    b = pl.program_id(0)
    v = valid_ref[b]

    @pl.when(v != 0)
    def _copy():
        reps_out[...] = reps_in[...]
        mask_out[...] = mask_in[...]

    @pl.when(v == 0)
    def _zero():
        reps_out[...] = jnp.zeros_like(reps_out)
        mask_out[...] = jnp.zeros_like(mask_out)


def kernel(token_reps, token_mask, valid_sentences):
    valid_i32 = valid_sentences.astype(jnp.int32)
    mask4 = token_mask.reshape(B, N, 1, L)

    reps_out, mask_out = pl.pallas_call(
        _select_kernel,
        grid=(B, S),
        in_specs=[
            pl.BlockSpec(memory_space=pltpu.SMEM),
            pl.BlockSpec((1, 1, L, D), lambda b, s: (b, s + 1, 0, 0)),
            pl.BlockSpec((1, 1, 1, L), lambda b, s: (b, s + 1, 0, 0)),
        ],
        out_specs=[
            pl.BlockSpec((1, 1, L, D), lambda b, s: (b, s, 0, 0)),
            pl.BlockSpec((1, 1, 1, L), lambda b, s: (b, s, 0, 0)),
        ],
        out_shape=[
            jax.ShapeDtypeStruct((B, S, L, D), jnp.float32),
            jax.ShapeDtypeStruct((B, S, 1, L), jnp.float32),
        ],
    )(valid_i32, token_reps, mask4)

    return reps_out, mask_out.reshape(B, S, L)
